# baseline (speedup 1.0000x reference)
import jax
import jax.numpy as jnp
from jax import lax
from jax.experimental import pallas as pl
from jax.experimental.pallas import tpu as pltpu
from jax.experimental.pallas import tpu_sc as plsc

F = 128
ROW = 4 * F
GROW = 6 * F
TILE = 96
K = 16
NSUB = 16
NSC = 2
NW = NSC * NSUB
NBUF = 4


def _mlp(q2p, W1, b1, W2, b2):
    npad = q2p.shape[0]
    bm = 512

    def body(q_ref, w1_ref, b1_ref, w2_ref, b2_ref, o_ref):
        h = jnp.dot(q_ref[:], w1_ref[:], preferred_element_type=jnp.float32)
        h = h + b1_ref[:]
        h = h * jax.nn.sigmoid(h)
        o_ref[:] = jnp.dot(h, w2_ref[:], preferred_element_type=jnp.float32) + b2_ref[:]

    return pl.pallas_call(
        body,
        grid=(npad // bm,),
        in_specs=[
            pl.BlockSpec((bm, F), lambda i: (i, 0)),
            pl.BlockSpec((F, F), lambda i: (0, 0)),
            pl.BlockSpec((1, F), lambda i: (0, 0)),
            pl.BlockSpec((F, 3 * F), lambda i: (0, 0)),
            pl.BlockSpec((1, 3 * F), lambda i: (0, 0)),
        ],
        out_specs=pl.BlockSpec((bm, 3 * F), lambda i: (i, 0)),
        out_shape=jax.ShapeDtypeStruct((npad, 3 * F), jnp.float32),
    )(q2p, W1, b1.reshape(1, F), W2, b2.reshape(1, 3 * F))


def _edge_kernel(ntiles, npad, nstarts):
    rounds = ntiles // NW

    def body(xmu_hbm, wij_hbm, auxi_hbm, auxd_hbm, qmu_hbm, st_hbm,
             out_hbm, acc_v, st_v, auxi_v, auxd_v, sidx_v, wij_v, xmu_v,
             sem_lin, sem_g):
        c = lax.axis_index("c")
        s = lax.axis_index("s")
        wid = s * NSC + c
        pltpu.sync_copy(st_hbm, st_v.at[pl.ds(0, nstarts)])
        lanes = lax.iota(jnp.int32, 16)

        def tile_body(r, carry):
            t = r * NW + wid
            base = t * TILE
            pltpu.sync_copy(qmu_hbm.at[pl.ds(base * ROW, TILE * ROW)],
                            acc_v.at[pl.ds(0, TILE * ROW)])
            stw = st_v[pl.ds(t, 16)]
            start = stw[0]
            end = stw[1]
            a16 = (start // K) * K
            nch = (end - a16 + (K - 1)) // K

            def lin_descs(i):
                b = lax.rem(i, NBUF)
                e0 = a16 + i * K
                return [
                    pltpu.make_async_copy(auxi_hbm.at[pl.ds(e0 * 2, K * 2)],
                                          auxi_v.at[b], sem_lin.at[b]),
                    pltpu.make_async_copy(auxd_hbm.at[pl.ds(e0 * 3, K * 3)],
                                          auxd_v.at[b, pl.ds(0, K * 3)],
                                          sem_lin.at[b]),
                    pltpu.make_async_copy(wij_hbm.at[pl.ds(e0 * (3 * F), K * 3 * F)],
                                          wij_v.at[b], sem_lin.at[b]),
                ]

            def g_descs(i):
                b = lax.rem(i, NBUF)
                return [
                    pltpu.make_async_copy(xmu_hbm.at[auxi_v.at[b, pl.ds(K, K)]],
                                          xmu_v.at[b], sem_g.at[b]),
                ]

            def prep(i):
                b = lax.rem(i, NBUF)
                e0 = a16 + i * K
                iidx = auxi_v[b, pl.ds(0, K)]
                evec = e0 + lanes
                valid = (evec >= start) & (evec < end)
                sidx_v[b, pl.ds(0, K)] = jnp.where(valid, iidx - base, TILE)

            def flush(prev_rel, regs):
                ao = prev_rel * ROW
                for j in range(32):
                    plsc.addupdate(acc_v.at[pl.ds(ao + j * 16, 16)], regs[j])

            def compute(i, carry3):
                b = lax.rem(i, NBUF)

                def edge_body(k, ec):
                    prev_rel, regs = ec
                    dvs = (jnp.full((16,), auxd_v[b, pl.ds(k, 16)][0], jnp.float32),
                           jnp.full((16,), auxd_v[b, pl.ds(K + k, 16)][0], jnp.float32),
                           jnp.full((16,), auxd_v[b, pl.ds(2 * K + k, 16)][0], jnp.float32))
                    rw = sidx_v[b, pl.ds(k, 16)]
                    rel = rw[0]
                    is_new = rel != prev_rel

                    @pl.when(is_new)
                    def _():
                        flush(prev_rel, regs)

                    ko = k * (3 * F)
                    nregs = [None] * 32
                    for r in range(8):
                        o = r * 16
                        cq = (wij_v[b, pl.ds(ko + o, 16)]
                              * xmu_v[b, k, pl.ds(o, 16)])
                        nregs[r] = jnp.where(is_new, cq, regs[r] + cq)
                        tR = (wij_v[b, pl.ds(ko + F + o, 16)]
                              * xmu_v[b, k, pl.ds(F + o, 16)])
                        tM = (wij_v[b, pl.ds(ko + 2 * F + o, 16)]
                              * xmu_v[b, k, pl.ds(2 * F + o, 16)])
                        for cc in range(3):
                            mj = xmu_v[b, k, pl.ds(3 * F + cc * F + o, 16)]
                            cm = tR * dvs[cc] + tM * mj
                            j = 8 + cc * 8 + r
                            nregs[j] = jnp.where(is_new, cm, regs[j] + cm)
                    return (rel, tuple(nregs))

                return lax.fori_loop(0, K, edge_body, carry3, unroll=4)

            for j in range(3):
                @pl.when(nch >= j + 1)
                def _(j=j):
                    for d in lin_descs(j):
                        d.start()

            for j in range(2):
                @pl.when(nch >= j + 1)
                def _(j=j):
                    for d in lin_descs(j):
                        d.wait()
                    prep(j)
                    for d in g_descs(j):
                        d.start()

            def chunk_body(i, carry2):
                @pl.when(i + 3 < nch)
                def _():
                    for d in lin_descs(i + 3):
                        d.start()

                @pl.when(i + 2 < nch)
                def _():
                    for d in lin_descs(i + 2):
                        d.wait()
                    prep(i + 2)
                    for d in g_descs(i + 2):
                        d.start()

                for d in g_descs(i):
                    d.wait()
                return compute(i, carry2)

            zero = jnp.zeros((16,), jnp.float32)
            carry0 = (jnp.int32(TILE + 1), tuple(zero for _ in range(32)))
            prev_rel, regs = lax.fori_loop(0, nch, chunk_body, carry0)
            flush(prev_rel, regs)
            pltpu.sync_copy(acc_v.at[pl.ds(0, TILE * ROW)],
                            out_hbm.at[pl.ds(base * ROW, TILE * ROW)])
            return carry

        lax.fori_loop(0, rounds, tile_body, 0)

    mesh = plsc.VectorSubcoreMesh(core_axis_name="c", subcore_axis_name="s")
    return pl.kernel(
        body,
        out_type=jax.ShapeDtypeStruct((npad * ROW,), jnp.float32),
        mesh=mesh,
        scratch_types=[
            pltpu.VMEM(((TILE + 2) * ROW,), jnp.float32),
            pltpu.VMEM((nstarts + 16,), jnp.int32),
            pltpu.VMEM((NBUF, 2 * K), jnp.int32),
            pltpu.VMEM((NBUF, 4 * K), jnp.float32),
            pltpu.VMEM((NBUF, 32), jnp.int32),
            pltpu.VMEM((NBUF, K * 3 * F), jnp.float32),
            pltpu.VMEM((NBUF, K, GROW), jnp.float32),
            pltpu.SemaphoreType.DMA((NBUF,)),
            pltpu.SemaphoreType.DMA((NBUF,)),
        ],
    )


def kernel(q, mu, Wij, dir_ij, idx_i, idx_j, n_atoms, W1, b1, W2, b2):
    N = q.shape[0]
    E = idx_i.shape[0]
    if E % K != 0:
        raise NotImplementedError("edge count must be a multiple of 16")
    ntiles = -(-N // (TILE * NW)) * NW
    npad = ntiles * TILE
    nstarts = -(-(ntiles + 1) // 16) * 16

    q2 = q.reshape(N, F)
    mu2 = mu.reshape(N, 3 * F)
    npad_mlp = -(-N // 512) * 512
    q2p = jnp.pad(q2, ((0, npad_mlp - N), (0, 0)))
    x_tab = _mlp(q2p, W1, b1, W2, b2)
    xmu = jnp.concatenate([x_tab[:N], mu2], axis=1)

    qmu = jnp.pad(jnp.concatenate([q2, mu2], axis=1),
                  ((0, npad - N), (0, 0))).reshape(npad * ROW)
    starts = jnp.searchsorted(
        idx_i, jnp.arange(ntiles + 1, dtype=jnp.int32) * TILE, side="left"
    ).astype(jnp.int32)
    starts = jnp.pad(starts, (0, nstarts - (ntiles + 1)))

    auxi = jnp.stack([idx_i.reshape(E // K, K), idx_j.reshape(E // K, K)],
                     axis=1).reshape(E * 2)
    auxd = dir_ij.reshape(E // K, K, 3).transpose(0, 2, 1).reshape(E * 3)

    out = _edge_kernel(ntiles, npad, nstarts)(
        xmu, Wij.reshape(E * 3 * F), auxi, auxd, qmu, starts)

    out = out.reshape(npad, ROW)
    q_out = out[:N, :F].reshape(N, 1, F)
    mu_out = out[:N, F:].reshape(N, 3, F)
    return (q_out, mu_out)


# speedup vs baseline: 1.0059x; 1.0059x over previous
import jax
import jax.numpy as jnp
from jax import lax
from jax.experimental import pallas as pl
from jax.experimental.pallas import tpu as pltpu
from jax.experimental.pallas import tpu_sc as plsc

F = 128
ROW = 4 * F
GROW = 6 * F
TILE = 96
K = 16
NSUB = 16
NSC = 2
NW = NSC * NSUB
NBUF = 4


def _mlp(q2p, W1, b1, W2, b2):
    npad = q2p.shape[0]
    bm = 512

    def body(q_ref, w1_ref, b1_ref, w2_ref, b2_ref, o_ref):
        h = jnp.dot(q_ref[:], w1_ref[:], preferred_element_type=jnp.float32)
        h = h + b1_ref[:]
        h = h * jax.nn.sigmoid(h)
        o_ref[:] = jnp.dot(h, w2_ref[:], preferred_element_type=jnp.float32) + b2_ref[:]

    return pl.pallas_call(
        body,
        grid=(npad // bm,),
        in_specs=[
            pl.BlockSpec((bm, F), lambda i: (i, 0)),
            pl.BlockSpec((F, F), lambda i: (0, 0)),
            pl.BlockSpec((1, F), lambda i: (0, 0)),
            pl.BlockSpec((F, 3 * F), lambda i: (0, 0)),
            pl.BlockSpec((1, 3 * F), lambda i: (0, 0)),
        ],
        out_specs=pl.BlockSpec((bm, 3 * F), lambda i: (i, 0)),
        out_shape=jax.ShapeDtypeStruct((npad, 3 * F), jnp.float32),
    )(q2p, W1, b1.reshape(1, F), W2, b2.reshape(1, 3 * F))


def _edge_kernel(ntiles, npad, nstarts):
    rounds = ntiles // NW

    def body(x_hbm, mu_hbm, wij_hbm, auxi_hbm, auxd_hbm, qmu_hbm, st_hbm,
             out_hbm, acc_v, st_v, auxi_v, auxd_v, sidx_v, wij_v, xj_v, muj_v,
             sem_lin, sem_g):
        c = lax.axis_index("c")
        s = lax.axis_index("s")
        wid = s * NSC + c
        pltpu.sync_copy(st_hbm, st_v.at[pl.ds(0, nstarts)])
        lanes = lax.iota(jnp.int32, 16)

        def tile_body(r, carry):
            t = r * NW + wid
            base = t * TILE
            pltpu.sync_copy(qmu_hbm.at[pl.ds(base * ROW, TILE * ROW)],
                            acc_v.at[pl.ds(0, TILE * ROW)])
            stw = st_v[pl.ds(t, 16)]
            start = stw[0]
            end = stw[1]
            a16 = (start // K) * K
            nch = (end - a16 + (K - 1)) // K

            def lin_descs(i):
                b = lax.rem(i, NBUF)
                e0 = a16 + i * K
                return [
                    pltpu.make_async_copy(auxi_hbm.at[pl.ds(e0 * 2, K * 2)],
                                          auxi_v.at[b], sem_lin.at[b]),
                    pltpu.make_async_copy(auxd_hbm.at[pl.ds(e0 * 3, K * 3)],
                                          auxd_v.at[b, pl.ds(0, K * 3)],
                                          sem_lin.at[b]),
                    pltpu.make_async_copy(wij_hbm.at[pl.ds(e0 * (3 * F), K * 3 * F)],
                                          wij_v.at[b], sem_lin.at[b]),
                ]

            def g_descs(i):
                b = lax.rem(i, NBUF)
                return [
                    pltpu.make_async_copy(x_hbm.at[auxi_v.at[b, pl.ds(K, K)]],
                                          xj_v.at[b], sem_g.at[b]),
                    pltpu.make_async_copy(mu_hbm.at[auxi_v.at[b, pl.ds(K, K)]],
                                          muj_v.at[b], sem_g.at[b]),
                ]

            def prep(i):
                b = lax.rem(i, NBUF)
                e0 = a16 + i * K
                iidx = auxi_v[b, pl.ds(0, K)]
                evec = e0 + lanes
                valid = (evec >= start) & (evec < end)
                sidx_v[b, pl.ds(0, K)] = jnp.where(valid, iidx - base, TILE)

            def flush(prev_rel, regs):
                ao = prev_rel * ROW
                for j in range(32):
                    plsc.addupdate(acc_v.at[pl.ds(ao + j * 16, 16)], regs[j])

            def compute(i, carry3):
                b = lax.rem(i, NBUF)

                def edge_body(k, ec):
                    prev_rel, regs = ec
                    dvs = (jnp.full((16,), auxd_v[b, pl.ds(k, 16)][0], jnp.float32),
                           jnp.full((16,), auxd_v[b, pl.ds(K + k, 16)][0], jnp.float32),
                           jnp.full((16,), auxd_v[b, pl.ds(2 * K + k, 16)][0], jnp.float32))
                    rw = sidx_v[b, pl.ds(k, 16)]
                    rel = rw[0]
                    is_new = rel != prev_rel

                    @pl.when(is_new)
                    def _():
                        flush(prev_rel, regs)

                    ko = k * (3 * F)
                    nregs = [None] * 32
                    for r in range(8):
                        o = r * 16
                        cq = (wij_v[b, pl.ds(ko + o, 16)]
                              * xj_v[b, k, pl.ds(o, 16)])
                        nregs[r] = jnp.where(is_new, cq, regs[r] + cq)
                        tR = (wij_v[b, pl.ds(ko + F + o, 16)]
                              * xj_v[b, k, pl.ds(F + o, 16)])
                        tM = (wij_v[b, pl.ds(ko + 2 * F + o, 16)]
                              * xj_v[b, k, pl.ds(2 * F + o, 16)])
                        for cc in range(3):
                            mj = muj_v[b, k, pl.ds(cc * F + o, 16)]
                            cm = tR * dvs[cc] + tM * mj
                            j = 8 + cc * 8 + r
                            nregs[j] = jnp.where(is_new, cm, regs[j] + cm)
                    return (rel, tuple(nregs))

                return lax.fori_loop(0, K, edge_body, carry3, unroll=4)

            for j in range(3):
                @pl.when(nch >= j + 1)
                def _(j=j):
                    for d in lin_descs(j):
                        d.start()

            for j in range(2):
                @pl.when(nch >= j + 1)
                def _(j=j):
                    for d in lin_descs(j):
                        d.wait()
                    prep(j)
                    for d in g_descs(j):
                        d.start()

            def chunk_body(i, carry2):
                @pl.when(i + 3 < nch)
                def _():
                    for d in lin_descs(i + 3):
                        d.start()

                @pl.when(i + 2 < nch)
                def _():
                    for d in lin_descs(i + 2):
                        d.wait()
                    prep(i + 2)
                    for d in g_descs(i + 2):
                        d.start()

                for d in g_descs(i):
                    d.wait()
                return compute(i, carry2)

            zero = jnp.zeros((16,), jnp.float32)
            carry0 = (jnp.int32(TILE + 1), tuple(zero for _ in range(32)))
            prev_rel, regs = lax.fori_loop(0, nch, chunk_body, carry0)
            flush(prev_rel, regs)
            pltpu.sync_copy(acc_v.at[pl.ds(0, TILE * ROW)],
                            out_hbm.at[pl.ds(base * ROW, TILE * ROW)])
            return carry

        lax.fori_loop(0, rounds, tile_body, 0)

    mesh = plsc.VectorSubcoreMesh(core_axis_name="c", subcore_axis_name="s")
    return pl.kernel(
        body,
        out_type=jax.ShapeDtypeStruct((npad * ROW,), jnp.float32),
        mesh=mesh,
        scratch_types=[
            pltpu.VMEM(((TILE + 2) * ROW,), jnp.float32),
            pltpu.VMEM((nstarts + 16,), jnp.int32),
            pltpu.VMEM((NBUF, 2 * K), jnp.int32),
            pltpu.VMEM((NBUF, 4 * K), jnp.float32),
            pltpu.VMEM((NBUF, 32), jnp.int32),
            pltpu.VMEM((NBUF, K * 3 * F), jnp.float32),
            pltpu.VMEM((NBUF, K, 3 * F), jnp.float32),
            pltpu.VMEM((NBUF, K, 3 * F), jnp.float32),
            pltpu.SemaphoreType.DMA((NBUF,)),
            pltpu.SemaphoreType.DMA((NBUF,)),
        ],
    )


def kernel(q, mu, Wij, dir_ij, idx_i, idx_j, n_atoms, W1, b1, W2, b2):
    N = q.shape[0]
    E = idx_i.shape[0]
    if E % K != 0:
        raise NotImplementedError("edge count must be a multiple of 16")
    ntiles = -(-N // (TILE * NW)) * NW
    npad = ntiles * TILE
    nstarts = -(-(ntiles + 1) // 16) * 16

    q2 = q.reshape(N, F)
    mu2 = mu.reshape(N, 3 * F)
    npad_mlp = -(-N // 512) * 512
    q2p = jnp.pad(q2, ((0, npad_mlp - N), (0, 0)))
    x_tab = _mlp(q2p, W1, b1, W2, b2)

    qmu = jnp.pad(jnp.concatenate([q2, mu2], axis=1),
                  ((0, npad - N), (0, 0))).reshape(npad * ROW)
    starts = jnp.searchsorted(
        idx_i, jnp.arange(ntiles + 1, dtype=jnp.int32) * TILE, side="left"
    ).astype(jnp.int32)
    starts = jnp.pad(starts, (0, nstarts - (ntiles + 1)))

    auxi = jnp.stack([idx_i.reshape(E // K, K), idx_j.reshape(E // K, K)],
                     axis=1).reshape(E * 2)
    auxd = dir_ij.reshape(E // K, K, 3).transpose(0, 2, 1).reshape(E * 3)

    out = _edge_kernel(ntiles, npad, nstarts)(
        x_tab, mu2, Wij.reshape(E * 3 * F), auxi, auxd, qmu, starts)

    out = out.reshape(npad, ROW)
    q_out = out[:N, :F].reshape(N, 1, F)
    mu_out = out[:N, F:].reshape(N, 3, F)
    return (q_out, mu_out)


# speedup vs baseline: 1.0466x; 1.0405x over previous
import jax
import jax.numpy as jnp
from jax import lax
from jax.experimental import pallas as pl
from jax.experimental.pallas import tpu as pltpu
from jax.experimental.pallas import tpu_sc as plsc

F = 128
ROW = 4 * F
TILE = 96
K = 16
NSUB = 16
NSC = 2
NW = NSC * NSUB
NBUF = 4


def _mlp(q2p, W1, b1, W2, b2):
    npad = q2p.shape[0]
    bm = 512

    def body(q_ref, w1_ref, b1_ref, w2_ref, b2_ref, o_ref):
        h = jnp.dot(q_ref[:], w1_ref[:], preferred_element_type=jnp.float32)
        h = h + b1_ref[:]
        h = h * jax.nn.sigmoid(h)
        o_ref[:] = jnp.dot(h, w2_ref[:], preferred_element_type=jnp.float32) + b2_ref[:]

    return pl.pallas_call(
        body,
        grid=(npad // bm,),
        in_specs=[
            pl.BlockSpec((bm, F), lambda i: (i, 0)),
            pl.BlockSpec((F, F), lambda i: (0, 0)),
            pl.BlockSpec((1, F), lambda i: (0, 0)),
            pl.BlockSpec((F, 3 * F), lambda i: (0, 0)),
            pl.BlockSpec((1, 3 * F), lambda i: (0, 0)),
        ],
        out_specs=pl.BlockSpec((bm, 3 * F), lambda i: (i, 0)),
        out_shape=jax.ShapeDtypeStruct((npad, 3 * F), jnp.float32),
    )(q2p, W1, b1.reshape(1, F), W2, b2.reshape(1, 3 * F))


def _edge_kernel(ntiles, npad, nstarts):
    rounds = ntiles // NW

    def body(x_hbm, mu_hbm, wij_hbm, dir_hbm, ii_hbm, ij_hbm, qmu_hbm, st_hbm,
             out_hbm, acc_v, st_v, iidx_v, jidx_v, sidx_v, dir_v, wij_v, xj_v,
             muj_v, sem_lin, sem_g):
        c = lax.axis_index("c")
        s = lax.axis_index("s")
        wid = s * NSC + c
        pltpu.sync_copy(st_hbm, st_v.at[pl.ds(0, nstarts)])

        def tile_body(r, carry):
            t = r * NW + wid
            base = t * TILE
            pltpu.sync_copy(qmu_hbm.at[pl.ds(base * ROW, TILE * ROW)],
                            acc_v.at[pl.ds(0, TILE * ROW)])
            stw = st_v[pl.ds(t, 16)]
            start = stw[0]
            end = stw[1]
            a16 = (start // K) * K
            nch = (end - a16 + (K - 1)) // K

            def lin_descs(i):
                b = lax.rem(i, NBUF)
                e0 = a16 + i * K
                return [
                    pltpu.make_async_copy(ii_hbm.at[pl.ds(e0, K)],
                                          iidx_v.at[b, pl.ds(0, K)], sem_lin.at[b]),
                    pltpu.make_async_copy(ij_hbm.at[pl.ds(e0, K)],
                                          jidx_v.at[b], sem_lin.at[b]),
                    pltpu.make_async_copy(dir_hbm.at[pl.ds(3 * e0, 3 * K)],
                                          dir_v.at[b, pl.ds(0, 3 * K)], sem_lin.at[b]),
                    pltpu.make_async_copy(wij_hbm.at[pl.ds(e0 * (3 * F), K * 3 * F)],
                                          wij_v.at[b], sem_lin.at[b]),
                ]

            def g_descs(i):
                b = lax.rem(i, NBUF)
                return [
                    pltpu.make_async_copy(x_hbm.at[jidx_v.at[b]], xj_v.at[b],
                                          sem_g.at[b]),
                    pltpu.make_async_copy(mu_hbm.at[jidx_v.at[b]], muj_v.at[b],
                                          sem_g.at[b]),
                ]

            def prep(i):
                b = lax.rem(i, NBUF)
                e0 = a16 + i * K
                evec = e0 + lax.iota(jnp.int32, 16)
                valid = (evec >= start) & (evec < end)
                sidx_v[b, pl.ds(0, K)] = jnp.where(
                    valid, iidx_v[b, pl.ds(0, K)] - base, TILE)

            def flush(prev_rel, regs):
                ao = prev_rel * ROW
                for j in range(32):
                    plsc.addupdate(acc_v.at[pl.ds(ao + j * 16, 16)], regs[j])

            def compute(i, carry3):
                b = lax.rem(i, NBUF)

                def edge_body(k, ec):
                    prev_rel, regs = ec
                    dw = dir_v[b, pl.ds(3 * k, 16)]
                    dvs = (jnp.full((16,), dw[0], jnp.float32),
                           jnp.full((16,), dw[1], jnp.float32),
                           jnp.full((16,), dw[2], jnp.float32))
                    rw = sidx_v[b, pl.ds(k, 16)]
                    rel = rw[0]
                    is_new = rel != prev_rel

                    @pl.when(is_new)
                    def _():
                        flush(prev_rel, regs)

                    ko = k * (3 * F)
                    nregs = [None] * 32
                    for r in range(8):
                        o = r * 16
                        cq = (wij_v[b, pl.ds(ko + o, 16)]
                              * xj_v[b, k, pl.ds(o, 16)])
                        nregs[r] = jnp.where(is_new, cq, regs[r] + cq)
                        tR = (wij_v[b, pl.ds(ko + F + o, 16)]
                              * xj_v[b, k, pl.ds(F + o, 16)])
                        tM = (wij_v[b, pl.ds(ko + 2 * F + o, 16)]
                              * xj_v[b, k, pl.ds(2 * F + o, 16)])
                        for cc in range(3):
                            mj = muj_v[b, k, pl.ds(cc * F + o, 16)]
                            cm = tR * dvs[cc] + tM * mj
                            j = 8 + cc * 8 + r
                            nregs[j] = jnp.where(is_new, cm, regs[j] + cm)
                    return (rel, tuple(nregs))

                return lax.fori_loop(0, K, edge_body, carry3, unroll=4)

            for j in range(3):
                @pl.when(nch >= j + 1)
                def _(j=j):
                    for d in lin_descs(j):
                        d.start()

            for j in range(2):
                @pl.when(nch >= j + 1)
                def _(j=j):
                    for d in lin_descs(j):
                        d.wait()
                    prep(j)
                    for d in g_descs(j):
                        d.start()

            def chunk_body(i, carry2):
                @pl.when(i + 3 < nch)
                def _():
                    for d in lin_descs(i + 3):
                        d.start()

                @pl.when(i + 2 < nch)
                def _():
                    for d in lin_descs(i + 2):
                        d.wait()
                    prep(i + 2)
                    for d in g_descs(i + 2):
                        d.start()

                for d in g_descs(i):
                    d.wait()
                return compute(i, carry2)

            zero = jnp.zeros((16,), jnp.float32)
            carry0 = (jnp.int32(TILE + 1), tuple(zero for _ in range(32)))
            prev_rel, regs = lax.fori_loop(0, nch, chunk_body, carry0)
            flush(prev_rel, regs)
            pltpu.sync_copy(acc_v.at[pl.ds(0, TILE * ROW)],
                            out_hbm.at[pl.ds(base * ROW, TILE * ROW)])
            return carry

        lax.fori_loop(0, rounds, tile_body, 0)

    mesh = plsc.VectorSubcoreMesh(core_axis_name="c", subcore_axis_name="s")
    return pl.kernel(
        body,
        out_type=jax.ShapeDtypeStruct((npad * ROW,), jnp.float32),
        mesh=mesh,
        scratch_types=[
            pltpu.VMEM(((TILE + 2) * ROW,), jnp.float32),
            pltpu.VMEM((nstarts + 16,), jnp.int32),
            pltpu.VMEM((NBUF, 32), jnp.int32),
            pltpu.VMEM((NBUF, K), jnp.int32),
            pltpu.VMEM((NBUF, 32), jnp.int32),
            pltpu.VMEM((NBUF, 64), jnp.float32),
            pltpu.VMEM((NBUF, K * 3 * F), jnp.float32),
            pltpu.VMEM((NBUF, K, 3 * F), jnp.float32),
            pltpu.VMEM((NBUF, K, 3 * F), jnp.float32),
            pltpu.SemaphoreType.DMA((NBUF,)),
            pltpu.SemaphoreType.DMA((NBUF,)),
        ],
    )


def kernel(q, mu, Wij, dir_ij, idx_i, idx_j, n_atoms, W1, b1, W2, b2):
    N = q.shape[0]
    E = idx_i.shape[0]
    if E % K != 0:
        raise NotImplementedError("edge count must be a multiple of 16")
    ntiles = -(-N // (TILE * NW)) * NW
    npad = ntiles * TILE
    nstarts = -(-(ntiles + 1) // 16) * 16

    q2 = q.reshape(N, F)
    mu2 = mu.reshape(N, 3 * F)
    npad_mlp = -(-N // 512) * 512
    q2p = jnp.pad(q2, ((0, npad_mlp - N), (0, 0)))
    x_tab = _mlp(q2p, W1, b1, W2, b2)

    qmu = jnp.pad(jnp.concatenate([q2, mu2], axis=1),
                  ((0, npad - N), (0, 0))).reshape(npad * ROW)
    starts = jnp.searchsorted(
        idx_i, jnp.arange(ntiles + 1, dtype=jnp.int32) * TILE, side="left"
    ).astype(jnp.int32)
    starts = jnp.pad(starts, (0, nstarts - (ntiles + 1)))

    out = _edge_kernel(ntiles, npad, nstarts)(
        x_tab, mu2, Wij.reshape(E * 3 * F), dir_ij.reshape(3 * E),
        idx_i, idx_j, qmu, starts)

    out = out.reshape(npad, ROW)
    q_out = out[:N, :F].reshape(N, 1, F)
    mu_out = out[:N, F:].reshape(N, 3, F)
    return (q_out, mu_out)


# speedup vs baseline: 1.0496x; 1.0029x over previous
import jax
import jax.numpy as jnp
from jax import lax
from jax.experimental import pallas as pl
from jax.experimental.pallas import tpu as pltpu
from jax.experimental.pallas import tpu_sc as plsc

F = 128
ROW = 4 * F
TILE = 96
K = 16
NSUB = 16
NSC = 2
NW = NSC * NSUB
NBUF = 4


def _mlp(q2p, W1, b1, W2, b2):
    npad = q2p.shape[0]
    bm = 512

    def body(q_ref, w1_ref, b1_ref, w2_ref, b2_ref, o_ref):
        h = jnp.dot(q_ref[:], w1_ref[:], preferred_element_type=jnp.float32)
        h = h + b1_ref[:]
        h = h * jax.nn.sigmoid(h)
        o_ref[:] = jnp.dot(h, w2_ref[:], preferred_element_type=jnp.float32) + b2_ref[:]

    return pl.pallas_call(
        body,
        grid=(npad // bm,),
        in_specs=[
            pl.BlockSpec((bm, F), lambda i: (i, 0)),
            pl.BlockSpec((F, F), lambda i: (0, 0)),
            pl.BlockSpec((1, F), lambda i: (0, 0)),
            pl.BlockSpec((F, 3 * F), lambda i: (0, 0)),
            pl.BlockSpec((1, 3 * F), lambda i: (0, 0)),
        ],
        out_specs=pl.BlockSpec((bm, 3 * F), lambda i: (i, 0)),
        out_shape=jax.ShapeDtypeStruct((npad, 3 * F), jnp.float32),
    )(q2p, W1, b1.reshape(1, F), W2, b2.reshape(1, 3 * F))


def _edge_kernel(ntiles, npad, nstarts):
    rounds = ntiles // NW

    def body(x_hbm, mu_hbm, wij_hbm, dir_hbm, ii_hbm, ij_hbm, qmu_hbm, st_hbm,
             out_hbm, acc_v, st_v, iidx_v, jidx_v, sidx_v, dir_v, wij_v, xj_v,
             muj_v, sem_lin, sem_g):
        c = lax.axis_index("c")
        s = lax.axis_index("s")
        wid = s * NSC + c
        pltpu.sync_copy(st_hbm, st_v.at[pl.ds(0, nstarts)])

        def tile_body(r, carry):
            t = r * NW + wid
            base = t * TILE
            pltpu.sync_copy(qmu_hbm.at[pl.ds(base * ROW, TILE * ROW)],
                            acc_v.at[pl.ds(0, TILE * ROW)])
            stw = st_v[pl.ds(t, 16)]
            start = stw[0]
            end = stw[1]
            a16 = (start // K) * K
            nch = (end - a16 + (K - 1)) // K

            def lin_descs(i):
                b = lax.rem(i, NBUF)
                e0 = a16 + i * K
                return [
                    pltpu.make_async_copy(ii_hbm.at[pl.ds(e0, K)],
                                          iidx_v.at[b, pl.ds(0, K)], sem_lin.at[b]),
                    pltpu.make_async_copy(ij_hbm.at[pl.ds(e0, K)],
                                          jidx_v.at[b], sem_lin.at[b]),
                    pltpu.make_async_copy(dir_hbm.at[pl.ds(3 * e0, 3 * K)],
                                          dir_v.at[b, pl.ds(0, 3 * K)], sem_lin.at[b]),
                    pltpu.make_async_copy(wij_hbm.at[pl.ds(e0 * (3 * F), K * 3 * F)],
                                          wij_v.at[b], sem_lin.at[b]),
                ]

            def g_descs(i):
                b = lax.rem(i, NBUF)
                return [
                    pltpu.make_async_copy(x_hbm.at[jidx_v.at[b]], xj_v.at[b],
                                          sem_g.at[b]),
                    pltpu.make_async_copy(mu_hbm.at[jidx_v.at[b]], muj_v.at[b],
                                          sem_g.at[b]),
                ]

            def prep(i):
                b = lax.rem(i, NBUF)
                e0 = a16 + i * K
                evec = e0 + lax.iota(jnp.int32, 16)
                valid = (evec >= start) & (evec < end)
                sidx_v[b, pl.ds(0, K)] = jnp.where(
                    valid, iidx_v[b, pl.ds(0, K)] - base, TILE)

            def flush(prev_rel, regs):
                ao = prev_rel * ROW
                for j in range(32):
                    plsc.addupdate(acc_v.at[pl.ds(ao + j * 16, 16)], regs[j])

            def compute(i, carry3):
                b = lax.rem(i, NBUF)

                def edge_body(k, ec):
                    prev_rel, regs = ec
                    dw = dir_v[b, pl.ds(3 * k, 16)]
                    dvs = (jnp.full((16,), dw[0], jnp.float32),
                           jnp.full((16,), dw[1], jnp.float32),
                           jnp.full((16,), dw[2], jnp.float32))
                    rw = sidx_v[b, pl.ds(k, 16)]
                    rel = rw[0]
                    is_new = rel != prev_rel

                    @pl.when(is_new)
                    def _():
                        flush(prev_rel, regs)

                    keep = jnp.full((16,), jnp.where(is_new, 0.0, 1.0),
                                    jnp.float32)
                    ko = k * (3 * F)
                    nregs = [None] * 32
                    for r in range(8):
                        o = r * 16
                        cq = (wij_v[b, pl.ds(ko + o, 16)]
                              * xj_v[b, k, pl.ds(o, 16)])
                        nregs[r] = regs[r] * keep + cq
                        tR = (wij_v[b, pl.ds(ko + F + o, 16)]
                              * xj_v[b, k, pl.ds(F + o, 16)])
                        tM = (wij_v[b, pl.ds(ko + 2 * F + o, 16)]
                              * xj_v[b, k, pl.ds(2 * F + o, 16)])
                        for cc in range(3):
                            mj = muj_v[b, k, pl.ds(cc * F + o, 16)]
                            cm = tR * dvs[cc] + tM * mj
                            j = 8 + cc * 8 + r
                            nregs[j] = regs[j] * keep + cm
                    return (rel, tuple(nregs))

                return lax.fori_loop(0, K, edge_body, carry3, unroll=4)

            for j in range(3):
                @pl.when(nch >= j + 1)
                def _(j=j):
                    for d in lin_descs(j):
                        d.start()

            for j in range(2):
                @pl.when(nch >= j + 1)
                def _(j=j):
                    for d in lin_descs(j):
                        d.wait()
                    prep(j)
                    for d in g_descs(j):
                        d.start()

            def chunk_body(i, carry2):
                @pl.when(i + 3 < nch)
                def _():
                    for d in lin_descs(i + 3):
                        d.start()

                @pl.when(i + 2 < nch)
                def _():
                    for d in lin_descs(i + 2):
                        d.wait()
                    prep(i + 2)
                    for d in g_descs(i + 2):
                        d.start()

                for d in g_descs(i):
                    d.wait()
                return compute(i, carry2)

            zero = jnp.zeros((16,), jnp.float32)
            carry0 = (jnp.int32(TILE + 1), tuple(zero for _ in range(32)))
            prev_rel, regs = lax.fori_loop(0, nch, chunk_body, carry0)
            flush(prev_rel, regs)
            pltpu.sync_copy(acc_v.at[pl.ds(0, TILE * ROW)],
                            out_hbm.at[pl.ds(base * ROW, TILE * ROW)])
            return carry

        lax.fori_loop(0, rounds, tile_body, 0)

    mesh = plsc.VectorSubcoreMesh(core_axis_name="c", subcore_axis_name="s")
    return pl.kernel(
        body,
        out_type=jax.ShapeDtypeStruct((npad * ROW,), jnp.float32),
        mesh=mesh,
        scratch_types=[
            pltpu.VMEM(((TILE + 2) * ROW,), jnp.float32),
            pltpu.VMEM((nstarts + 16,), jnp.int32),
            pltpu.VMEM((NBUF, 32), jnp.int32),
            pltpu.VMEM((NBUF, K), jnp.int32),
            pltpu.VMEM((NBUF, 32), jnp.int32),
            pltpu.VMEM((NBUF, 64), jnp.float32),
            pltpu.VMEM((NBUF, K * 3 * F), jnp.float32),
            pltpu.VMEM((NBUF, K, 3 * F), jnp.float32),
            pltpu.VMEM((NBUF, K, 3 * F), jnp.float32),
            pltpu.SemaphoreType.DMA((NBUF,)),
            pltpu.SemaphoreType.DMA((NBUF,)),
        ],
    )


def kernel(q, mu, Wij, dir_ij, idx_i, idx_j, n_atoms, W1, b1, W2, b2):
    N = q.shape[0]
    E = idx_i.shape[0]
    if E % K != 0:
        raise NotImplementedError("edge count must be a multiple of 16")
    ntiles = -(-N // (TILE * NW)) * NW
    npad = ntiles * TILE
    nstarts = -(-(ntiles + 1) // 16) * 16

    q2 = q.reshape(N, F)
    mu2 = mu.reshape(N, 3 * F)
    npad_mlp = -(-N // 512) * 512
    q2p = jnp.pad(q2, ((0, npad_mlp - N), (0, 0)))
    x_tab = _mlp(q2p, W1, b1, W2, b2)

    qmu = jnp.pad(jnp.concatenate([q2, mu2], axis=1),
                  ((0, npad - N), (0, 0))).reshape(npad * ROW)
    starts = jnp.searchsorted(
        idx_i, jnp.arange(ntiles + 1, dtype=jnp.int32) * TILE, side="left"
    ).astype(jnp.int32)
    starts = jnp.pad(starts, (0, nstarts - (ntiles + 1)))

    out = _edge_kernel(ntiles, npad, nstarts)(
        x_tab, mu2, Wij.reshape(E * 3 * F), dir_ij.reshape(3 * E),
        idx_i, idx_j, qmu, starts)

    out = out.reshape(npad, ROW)
    q_out = out[:N, :F].reshape(N, 1, F)
    mu_out = out[:N, F:].reshape(N, 3, F)
    return (q_out, mu_out)


# speedup vs baseline: 1.1528x; 1.0982x over previous
import jax
import jax.numpy as jnp
from jax import lax
from jax.experimental import pallas as pl
from jax.experimental.pallas import tpu as pltpu
from jax.experimental.pallas import tpu_sc as plsc

F = 128
ROW = 4 * F
TILE = 105
K = 16
NSUB = 16
NSC = 2
NW = NSC * NSUB
NBUF = 4


def _mlp(q2p, W1, b1, W2, b2):
    npad = q2p.shape[0]
    bm = 512

    def body(q_ref, w1_ref, b1_ref, w2_ref, b2_ref, o_ref):
        h = jnp.dot(q_ref[:], w1_ref[:], preferred_element_type=jnp.float32)
        h = h + b1_ref[:]
        h = h * jax.nn.sigmoid(h)
        o_ref[:] = jnp.dot(h, w2_ref[:], preferred_element_type=jnp.float32) + b2_ref[:]

    return pl.pallas_call(
        body,
        grid=(npad // bm,),
        in_specs=[
            pl.BlockSpec((bm, F), lambda i: (i, 0)),
            pl.BlockSpec((F, F), lambda i: (0, 0)),
            pl.BlockSpec((1, F), lambda i: (0, 0)),
            pl.BlockSpec((F, 3 * F), lambda i: (0, 0)),
            pl.BlockSpec((1, 3 * F), lambda i: (0, 0)),
        ],
        out_specs=pl.BlockSpec((bm, 3 * F), lambda i: (i, 0)),
        out_shape=jax.ShapeDtypeStruct((npad, 3 * F), jnp.float32),
    )(q2p, W1, b1.reshape(1, F), W2, b2.reshape(1, 3 * F))


def _edge_kernel(ntiles, npad, nstarts):
    rounds = ntiles // NW

    def body(x_hbm, mu_hbm, wij_hbm, dir_hbm, ii_hbm, ij_hbm, qmu_hbm, st_hbm,
             out_hbm, acc_v, st_v, iidx_v, jidx_v, sidx_v, dir_v, wij_v, xj_v,
             muj_v, sem_lin, sem_g):
        c = lax.axis_index("c")
        s = lax.axis_index("s")
        wid = s * NSC + c
        pltpu.sync_copy(st_hbm, st_v.at[pl.ds(0, nstarts)])

        def tile_body(r, carry):
            t = r * NW + wid
            base = t * TILE
            pltpu.sync_copy(qmu_hbm.at[pl.ds(base * ROW, TILE * ROW)],
                            acc_v.at[pl.ds(0, TILE * ROW)])
            stw = st_v[pl.ds(t, 16)]
            start = stw[0]
            end = stw[1]
            a16 = (start // K) * K
            nch = (end - a16 + (K - 1)) // K

            def lin_descs(i):
                b = lax.rem(i, NBUF)
                e0 = a16 + i * K
                return [
                    pltpu.make_async_copy(ii_hbm.at[pl.ds(e0, K)],
                                          iidx_v.at[b, pl.ds(0, K)], sem_lin.at[b]),
                    pltpu.make_async_copy(ij_hbm.at[pl.ds(e0, K)],
                                          jidx_v.at[b], sem_lin.at[b]),
                    pltpu.make_async_copy(dir_hbm.at[pl.ds(3 * e0, 3 * K)],
                                          dir_v.at[b, pl.ds(0, 3 * K)], sem_lin.at[b]),
                    pltpu.make_async_copy(wij_hbm.at[pl.ds(e0 * (3 * F), K * 3 * F)],
                                          wij_v.at[b], sem_lin.at[b]),
                ]

            def g_descs(i):
                b = lax.rem(i, NBUF)
                return [
                    pltpu.make_async_copy(x_hbm.at[jidx_v.at[b]], xj_v.at[b],
                                          sem_g.at[b]),
                    pltpu.make_async_copy(mu_hbm.at[jidx_v.at[b]], muj_v.at[b],
                                          sem_g.at[b]),
                ]

            def prep(i):
                b = lax.rem(i, NBUF)
                e0 = a16 + i * K
                evec = e0 + lax.iota(jnp.int32, 16)
                valid = (evec >= start) & (evec < end)
                sidx_v[b, pl.ds(0, K)] = jnp.where(
                    valid, iidx_v[b, pl.ds(0, K)] - base, TILE)

            def flush(prev_rel, regs):
                ao = prev_rel * ROW
                for j in range(32):
                    plsc.addupdate(acc_v.at[pl.ds(ao + j * 16, 16)], regs[j])

            def compute(i, carry3):
                b = lax.rem(i, NBUF)

                def edge_body(k, ec):
                    prev_rel, regs = ec
                    dw = dir_v[b, pl.ds(3 * k, 16)]
                    dvs = (jnp.full((16,), dw[0], jnp.float32),
                           jnp.full((16,), dw[1], jnp.float32),
                           jnp.full((16,), dw[2], jnp.float32))
                    rw = sidx_v[b, pl.ds(k, 16)]
                    rel = rw[0]
                    is_new = rel != prev_rel

                    @pl.when(is_new)
                    def _():
                        flush(prev_rel, regs)

                    keep = jnp.full((16,), jnp.where(is_new, 0.0, 1.0),
                                    jnp.float32)
                    ko = k * (3 * F)
                    nregs = [None] * 32
                    for r in range(8):
                        o = r * 16
                        cq = (wij_v[b, pl.ds(ko + o, 16)]
                              * xj_v[b, k, pl.ds(o, 16)])
                        nregs[r] = regs[r] * keep + cq
                        tR = (wij_v[b, pl.ds(ko + F + o, 16)]
                              * xj_v[b, k, pl.ds(F + o, 16)])
                        tM = (wij_v[b, pl.ds(ko + 2 * F + o, 16)]
                              * xj_v[b, k, pl.ds(2 * F + o, 16)])
                        for cc in range(3):
                            mj = muj_v[b, k, pl.ds(cc * F + o, 16)]
                            cm = tR * dvs[cc] + tM * mj
                            j = 8 + cc * 8 + r
                            nregs[j] = regs[j] * keep + cm
                    return (rel, tuple(nregs))

                return lax.fori_loop(0, K, edge_body, carry3, unroll=4)

            for j in range(3):
                @pl.when(nch >= j + 1)
                def _(j=j):
                    for d in lin_descs(j):
                        d.start()

            for j in range(2):
                @pl.when(nch >= j + 1)
                def _(j=j):
                    for d in lin_descs(j):
                        d.wait()
                    prep(j)
                    for d in g_descs(j):
                        d.start()

            def chunk_body(i, carry2):
                @pl.when(i + 3 < nch)
                def _():
                    for d in lin_descs(i + 3):
                        d.start()

                @pl.when(i + 2 < nch)
                def _():
                    for d in lin_descs(i + 2):
                        d.wait()
                    prep(i + 2)
                    for d in g_descs(i + 2):
                        d.start()

                for d in g_descs(i):
                    d.wait()
                return compute(i, carry2)

            zero = jnp.zeros((16,), jnp.float32)
            carry0 = (jnp.int32(TILE + 1), tuple(zero for _ in range(32)))
            prev_rel, regs = lax.fori_loop(0, nch, chunk_body, carry0)
            flush(prev_rel, regs)
            pltpu.sync_copy(acc_v.at[pl.ds(0, TILE * ROW)],
                            out_hbm.at[pl.ds(base * ROW, TILE * ROW)])
            return carry

        lax.fori_loop(0, rounds, tile_body, 0)

    mesh = plsc.VectorSubcoreMesh(core_axis_name="c", subcore_axis_name="s")
    return pl.kernel(
        body,
        out_type=jax.ShapeDtypeStruct((npad * ROW,), jnp.float32),
        mesh=mesh,
        scratch_types=[
            pltpu.VMEM(((TILE + 2) * ROW,), jnp.float32),
            pltpu.VMEM((nstarts + 16,), jnp.int32),
            pltpu.VMEM((NBUF, 32), jnp.int32),
            pltpu.VMEM((NBUF, K), jnp.int32),
            pltpu.VMEM((NBUF, 32), jnp.int32),
            pltpu.VMEM((NBUF, 64), jnp.float32),
            pltpu.VMEM((NBUF, K * 3 * F), jnp.float32),
            pltpu.VMEM((NBUF, K, 3 * F), jnp.float32),
            pltpu.VMEM((NBUF, K, 3 * F), jnp.float32),
            pltpu.SemaphoreType.DMA((NBUF,)),
            pltpu.SemaphoreType.DMA((NBUF,)),
        ],
    )


def kernel(q, mu, Wij, dir_ij, idx_i, idx_j, n_atoms, W1, b1, W2, b2):
    N = q.shape[0]
    E = idx_i.shape[0]
    if E % K != 0:
        raise NotImplementedError("edge count must be a multiple of 16")
    ntiles = -(-N // (TILE * NW)) * NW
    npad = ntiles * TILE
    nstarts = -(-(ntiles + 1) // 16) * 16

    q2 = q.reshape(N, F)
    mu2 = mu.reshape(N, 3 * F)
    npad_mlp = -(-N // 512) * 512
    q2p = jnp.pad(q2, ((0, npad_mlp - N), (0, 0)))
    x_tab = _mlp(q2p, W1, b1, W2, b2)

    qmu = jnp.pad(jnp.concatenate([q2, mu2], axis=1),
                  ((0, npad - N), (0, 0))).reshape(npad * ROW)
    starts = jnp.searchsorted(
        idx_i, jnp.arange(ntiles + 1, dtype=jnp.int32) * TILE, side="left"
    ).astype(jnp.int32)
    starts = jnp.pad(starts, (0, nstarts - (ntiles + 1)))

    out = _edge_kernel(ntiles, npad, nstarts)(
        x_tab, mu2, Wij.reshape(E * 3 * F), dir_ij.reshape(3 * E),
        idx_i, idx_j, qmu, starts)

    out = out.reshape(npad, ROW)
    q_out = out[:N, :F].reshape(N, 1, F)
    mu_out = out[:N, F:].reshape(N, 3, F)
    return (q_out, mu_out)
